# Initial kernel scaffold; baseline (speedup 1.0000x reference)
import functools
import math

import jax
import jax.numpy as jnp
from jax import lax
from jax.experimental import pallas as pl
from jax.experimental.pallas import tpu as pltpu
from jax.experimental.pallas import tpu_sc as plsc

N_NODES = 10000
N_EDGES = 320000
D = 128
NUM_CLASSES = 40
NUM_LAYERS = 4
C_MIN = 0.2
C_MAX = 1.0
BETA = 0.1
GAMMA = 1.0
RW = C_MIN - BETA
A_AGG = 1.0 - RW - BETA

NC = 2
NS = 16
NW = NC * NS
L = 16
CHUNK = 128
N_CHUNKS = N_EDGES // CHUNK
LOOPS = (N_CHUNKS + NW - 1) // NW
ROWS_PER_TILE = N_NODES // NS

_sc_mesh = plsc.VectorSubcoreMesh(
    core_axis_name="c", subcore_axis_name="s", num_cores=NC, num_subcores=NS)


def _worker_id():
    return lax.axis_index("s") * NC + lax.axis_index("c")


@functools.partial(
    pl.kernel,
    out_type=jax.ShapeDtypeStruct((NC, N_NODES), jnp.float32),
    mesh=_sc_mesh,
    scratch_types=[
        pltpu.VMEM((CHUNK,), jnp.int32),
        pltpu.VMEM((CHUNK,), jnp.float32),
        pltpu.VMEM((1024,), jnp.float32),
        pltpu.VMEM_SHARED((N_NODES,), jnp.float32),
    ],
)
def _deg_kernel(row_hbm, w_hbm, deg_hbm, ridx, wbuf, zbuf, deg_sp):
    c = lax.axis_index("c")
    s = lax.axis_index("s")
    w = _worker_id()

    def zb(i, _):
        zbuf[pl.ds(i * L, L)] = jnp.zeros((L,), jnp.float32)
        return 0
    lax.fori_loop(0, 1024 // L, zb, 0)

    @pl.when(s < 10)
    def _():
        pltpu.sync_copy(zbuf.at[pl.ds(0, 1000)], deg_sp.at[pl.ds(s * 1000, 1000)])
    plsc.subcore_barrier()

    def body(k, _):
        ch = w + k * NW
        @pl.when(ch < N_CHUNKS)
        def _():
            base = ch * CHUNK
            pltpu.sync_copy(row_hbm.at[pl.ds(base, CHUNK)], ridx)
            pltpu.sync_copy(w_hbm.at[pl.ds(base, CHUNK)], wbuf)
            pltpu.sync_copy(wbuf, deg_sp.at[ridx], add=True)
        return 0
    lax.fori_loop(0, LOOPS, body, 0)
    plsc.subcore_barrier()

    @pl.when(s < 10)
    def _():
        pltpu.sync_copy(deg_sp.at[pl.ds(s * 1000, 1000)],
                        deg_hbm.at[c, pl.ds(s * 1000, 1000)])


@functools.partial(
    pl.kernel,
    out_type=jax.ShapeDtypeStruct((N_EDGES,), jnp.float32),
    mesh=_sc_mesh,
    scratch_types=[
        pltpu.VMEM((N_NODES,), jnp.float32),
        pltpu.VMEM((N_NODES,), jnp.float32),
        pltpu.VMEM((CHUNK,), jnp.int32),
        pltpu.VMEM((CHUNK,), jnp.int32),
        pltpu.VMEM((CHUNK,), jnp.float32),
        pltpu.VMEM((CHUNK,), jnp.float32),
    ],
)
def _norm_kernel(deg_hbm, row_hbm, col_hbm, w_hbm, norm_hbm,
                 dinv, dtmp, ridx, cidx, wbuf, nbuf):
    w = _worker_id()
    pltpu.sync_copy(deg_hbm.at[0], dinv)
    pltpu.sync_copy(deg_hbm.at[1], dtmp)

    def newton(i, _):
        sl = pl.ds(i * L, L)
        d = dinv[sl] + dtmp[sl]
        bits = plsc.bitcast(d, jnp.int32)
        y = plsc.bitcast(jnp.int32(0x5F3759DF) - (bits >> 1), jnp.float32)
        for _ in range(3):
            y = y * (1.5 - 0.5 * d * y * y)
        dinv[sl] = jnp.where(d > 0.0, y, 0.0)
        return 0
    lax.fori_loop(0, N_NODES // L, newton, 0)

    def body(k, _):
        ch = w + k * NW
        @pl.when(ch < N_CHUNKS)
        def _():
            base = ch * CHUNK
            pltpu.sync_copy(row_hbm.at[pl.ds(base, CHUNK)], ridx)
            pltpu.sync_copy(col_hbm.at[pl.ds(base, CHUNK)], cidx)
            pltpu.sync_copy(w_hbm.at[pl.ds(base, CHUNK)], wbuf)
            for g in range(CHUNK // L):
                sl = pl.ds(g * L, L)
                dr = plsc.load_gather(dinv, [ridx[sl]])
                dc = plsc.load_gather(dinv, [cidx[sl]])
                nbuf[sl] = wbuf[sl] * dr * dc
            pltpu.sync_copy(nbuf, norm_hbm.at[pl.ds(base, CHUNK)])
        return 0
    lax.fori_loop(0, LOOPS, body, 0)


@functools.partial(
    pl.kernel,
    out_type=jax.ShapeDtypeStruct((NC, N_NODES, D), jnp.float32),
    mesh=_sc_mesh,
    scratch_types=[
        pltpu.VMEM((CHUNK,), jnp.int32),
        pltpu.VMEM((CHUNK,), jnp.int32),
        pltpu.VMEM((CHUNK,), jnp.float32),
        pltpu.VMEM((CHUNK, D), jnp.float32),
        pltpu.VMEM_SHARED((N_NODES, D), jnp.float32),
        pltpu.SemaphoreType.DMA,
    ],
)
def _msg_kernel(h_hbm, row_hbm, col_hbm, norm_hbm, out_hbm,
                ridx, cidx, nrm, rows, agg, sem):
    c = lax.axis_index("c")
    s = lax.axis_index("s")
    w = _worker_id()

    def zb(i, _):
        def zf(f, _):
            rows[i, pl.ds(f * L, L)] = jnp.zeros((L,), jnp.float32)
            return 0
        lax.fori_loop(0, D // L, zf, 0)
        return 0
    lax.fori_loop(0, CHUNK, zb, 0)
    base_row = s * ROWS_PER_TILE
    for k in range(5):
        n = 128 if k < 4 else ROWS_PER_TILE - 4 * 128
        pltpu.sync_copy(rows.at[pl.ds(0, n)],
                        agg.at[pl.ds(base_row + k * 128, n)])
    plsc.subcore_barrier()

    def body(k, _):
        ch = w + k * NW
        @pl.when(ch < N_CHUNKS)
        def _():
            base = ch * CHUNK
            pltpu.sync_copy(row_hbm.at[pl.ds(base, CHUNK)], ridx)
            pltpu.sync_copy(col_hbm.at[pl.ds(base, CHUNK)], cidx)
            pltpu.sync_copy(norm_hbm.at[pl.ds(base, CHUNK)], nrm)
            pltpu.async_copy(h_hbm.at[ridx], rows, sem).wait()

            def e_body(e, _):
                sc_ = nrm[e]
                for f in range(D // L):
                    sl = pl.ds(f * L, L)
                    rows[e, sl] = rows[e, sl] * sc_
                return 0
            lax.fori_loop(0, CHUNK, e_body, 0)
            pltpu.sync_copy(rows, agg.at[cidx], add=True)
        return 0
    lax.fori_loop(0, LOOPS, body, 0)
    plsc.subcore_barrier()

    pltpu.sync_copy(agg.at[pl.ds(base_row, ROWS_PER_TILE)],
                    out_hbm.at[c, pl.ds(base_row, ROWS_PER_TILE)])


_BLK = 1250
_GRID = N_NODES // _BLK


def _dot(a, b):
    return jnp.dot(a, b, preferred_element_type=jnp.float32,
                   precision=lax.Precision.HIGHEST)


def _tc_in_body(x_ref, w_ref, b_ref, o_ref):
    o_ref[...] = jnp.maximum(_dot(x_ref[...], w_ref[...]) + b_ref[...], 0.0)


_tc_in = pl.pallas_call(
    _tc_in_body,
    grid=(_GRID,),
    in_specs=[
        pl.BlockSpec((_BLK, D), lambda i: (i, 0)),
        pl.BlockSpec((D, D), lambda i: (0, 0)),
        pl.BlockSpec((1, D), lambda i: (0, 0)),
    ],
    out_specs=pl.BlockSpec((_BLK, D), lambda i: (i, 0)),
    out_shape=jax.ShapeDtypeStruct((N_NODES, D), jnp.float32),
)


def _tc_combine_body(p_ref, h_ref, h0_ref, w_ref, o_ref):
    a = (A_AGG * (p_ref[0] + p_ref[1]) + RW * h_ref[...]
         + BETA * h0_ref[...])
    o_ref[...] = jnp.maximum(_dot(a, w_ref[...]), 0.0)


_tc_combine = pl.pallas_call(
    _tc_combine_body,
    grid=(_GRID,),
    in_specs=[
        pl.BlockSpec((NC, _BLK, D), lambda i: (0, i, 0)),
        pl.BlockSpec((_BLK, D), lambda i: (i, 0)),
        pl.BlockSpec((_BLK, D), lambda i: (i, 0)),
        pl.BlockSpec((D, D), lambda i: (0, 0)),
    ],
    out_specs=pl.BlockSpec((_BLK, D), lambda i: (i, 0)),
    out_shape=jax.ShapeDtypeStruct((N_NODES, D), jnp.float32),
)


def _tc_out_body(p_ref, h_ref, h0_ref, w4_ref, wo_ref, bo_ref, o_ref):
    a = (A_AGG * (p_ref[0] + p_ref[1]) + RW * h_ref[...]
         + BETA * h0_ref[...])
    h4 = jnp.maximum(_dot(a, w4_ref[...]), 0.0)
    logits = _dot(h4, wo_ref[...]) + bo_ref[...]
    m = jnp.max(logits, axis=1, keepdims=True)
    ex = jnp.exp(logits - m)
    lse = jnp.log(jnp.sum(ex, axis=1, keepdims=True)) + m
    o_ref[...] = logits - lse


_tc_out = pl.pallas_call(
    _tc_out_body,
    grid=(_GRID,),
    in_specs=[
        pl.BlockSpec((NC, _BLK, D), lambda i: (0, i, 0)),
        pl.BlockSpec((_BLK, D), lambda i: (i, 0)),
        pl.BlockSpec((_BLK, D), lambda i: (i, 0)),
        pl.BlockSpec((D, D), lambda i: (0, 0)),
        pl.BlockSpec((D, D), lambda i: (0, 0)),
        pl.BlockSpec((1, D), lambda i: (0, 0)),
    ],
    out_specs=pl.BlockSpec((_BLK, D), lambda i: (i, 0)),
    out_shape=jax.ShapeDtypeStruct((N_NODES, D), jnp.float32),
)


def _tc_lc_body(wg_ref, o_ref):
    r = lax.broadcasted_iota(jnp.int32, (D, D), 0)
    col = lax.broadcasted_iota(jnp.int32, (D, D), 1)
    eye = jnp.where(r == col, 1.0, 0.0).astype(jnp.float32)
    total = jnp.float32(0.0)
    for i in range(NUM_LAYERS):
        diff = wg_ref[i] - eye
        total = total + jnp.sqrt(jnp.sum(diff * diff))
    o_ref[...] = jnp.full((1, 1), total, jnp.float32)


_tc_lc = pl.pallas_call(
    _tc_lc_body,
    out_shape=jax.ShapeDtypeStruct((1, 1), jnp.float32),
)


def kernel(x, edge_index, edge_weight, W_in, b_in, W_gcn, W_out, b_out):
    row = edge_index[0]
    col = edge_index[1]

    h0 = _tc_in(x, W_in, b_in.reshape(1, D))
    degp = _deg_kernel(row, edge_weight)
    norm = _norm_kernel(degp, row, col, edge_weight)

    h = h0
    for i in range(NUM_LAYERS - 1):
        p = _msg_kernel(h, row, col, norm)
        h = _tc_combine(p, h, h0, W_gcn[i])

    p = _msg_kernel(h, row, col, norm)
    wo_pad = jnp.zeros((D, D), jnp.float32).at[:, :NUM_CLASSES].set(W_out)
    bo_pad = jnp.full((D,), -1e30, jnp.float32).at[:NUM_CLASSES].set(b_out)
    yfull = _tc_out(p, h, h0, W_gcn[NUM_LAYERS - 1], wo_pad,
                    bo_pad.reshape(1, D))
    y = yfull[:, :NUM_CLASSES]

    lc = _tc_lc(W_gcn)[0, 0] * GAMMA
    return (y, lc)


# speedup vs baseline: 8.6285x; 8.6285x over previous
import functools
import math

import jax
import jax.numpy as jnp
from jax import lax
from jax.experimental import pallas as pl
from jax.experimental.pallas import tpu as pltpu
from jax.experimental.pallas import tpu_sc as plsc

N_NODES = 10000
N_EDGES = 320000
D = 128
NUM_CLASSES = 40
NUM_LAYERS = 4
C_MIN = 0.2
C_MAX = 1.0
BETA = 0.1
GAMMA = 1.0
RW = C_MIN - BETA
A_AGG = 1.0 - RW - BETA

NC = 2
NS = 16
NW = NC * NS
L = 16
CHUNK = 128
N_CHUNKS = N_EDGES // CHUNK
LOOPS = (N_CHUNKS + NW - 1) // NW
ROWS_PER_TILE = N_NODES // NS

_sc_mesh = plsc.VectorSubcoreMesh(
    core_axis_name="c", subcore_axis_name="s", num_cores=NC, num_subcores=NS)


def _worker_id():
    return lax.axis_index("s") * NC + lax.axis_index("c")


@functools.partial(
    pl.kernel,
    out_type=jax.ShapeDtypeStruct((NC * N_NODES,), jnp.float32),
    mesh=_sc_mesh,
    compiler_params=pltpu.CompilerParams(needs_layout_passes=False),
    scratch_types=[
        pltpu.VMEM((CHUNK,), jnp.int32),
        pltpu.VMEM((CHUNK,), jnp.float32),
        pltpu.VMEM((1024,), jnp.float32),
        pltpu.VMEM_SHARED((N_NODES,), jnp.float32),
    ],
)
def _deg_kernel(row_hbm, w_hbm, deg_hbm, ridx, wbuf, zbuf, deg_sp):
    c = lax.axis_index("c")
    s = lax.axis_index("s")
    w = _worker_id()

    def zb(i, _):
        zbuf[pl.ds(i * L, L)] = jnp.zeros((L,), jnp.float32)
        return 0
    lax.fori_loop(0, 1024 // L, zb, 0)

    @pl.when(s < 10)
    def _():
        pltpu.sync_copy(zbuf.at[pl.ds(0, 1000)], deg_sp.at[pl.ds(s * 1000, 1000)])
    plsc.subcore_barrier()

    def body(k, _):
        ch = w + k * NW
        @pl.when(ch < N_CHUNKS)
        def _():
            base = ch * CHUNK
            pltpu.sync_copy(row_hbm.at[pl.ds(base, CHUNK)], ridx)
            pltpu.sync_copy(w_hbm.at[pl.ds(base, CHUNK)], wbuf)
            pltpu.sync_copy(wbuf, deg_sp.at[ridx], add=True)
        return 0
    lax.fori_loop(0, LOOPS, body, 0)
    plsc.subcore_barrier()

    @pl.when(s < 10)
    def _():
        pltpu.sync_copy(deg_sp.at[pl.ds(s * 1000, 1000)], zbuf.at[pl.ds(0, 1000)])
        pltpu.sync_copy(zbuf.at[pl.ds(0, 1000)],
                        deg_hbm.at[pl.ds(c * N_NODES + s * 1000, 1000)])


@functools.partial(
    pl.kernel,
    out_type=jax.ShapeDtypeStruct((N_EDGES,), jnp.float32),
    mesh=_sc_mesh,
    compiler_params=pltpu.CompilerParams(needs_layout_passes=False),
    scratch_types=[
        pltpu.VMEM((N_NODES,), jnp.float32),
        pltpu.VMEM((N_NODES,), jnp.float32),
        pltpu.VMEM((CHUNK,), jnp.int32),
        pltpu.VMEM((CHUNK,), jnp.int32),
        pltpu.VMEM((CHUNK,), jnp.float32),
        pltpu.VMEM((CHUNK,), jnp.float32),
    ],
)
def _norm_kernel(deg_hbm, row_hbm, col_hbm, w_hbm, norm_hbm,
                 dinv, dtmp, ridx, cidx, wbuf, nbuf):
    w = _worker_id()
    pltpu.sync_copy(deg_hbm.at[pl.ds(0, N_NODES)], dinv)
    pltpu.sync_copy(deg_hbm.at[pl.ds(N_NODES, N_NODES)], dtmp)

    def newton(i, _):
        sl = pl.ds(i * L, L)
        d = dinv[sl] + dtmp[sl]
        bits = lax.bitcast_convert_type(d, jnp.int32)
        y = lax.bitcast_convert_type(jnp.int32(0x5F3759DF) - (bits >> 1),
                                     jnp.float32)
        for _ in range(3):
            y = y * (1.5 - 0.5 * d * y * y)
        dinv[sl] = jnp.where(d > 0.0, y, 0.0)
        return 0
    lax.fori_loop(0, N_NODES // L, newton, 0)

    def body(k, _):
        ch = w + k * NW
        @pl.when(ch < N_CHUNKS)
        def _():
            base = ch * CHUNK
            pltpu.sync_copy(row_hbm.at[pl.ds(base, CHUNK)], ridx)
            pltpu.sync_copy(col_hbm.at[pl.ds(base, CHUNK)], cidx)
            pltpu.sync_copy(w_hbm.at[pl.ds(base, CHUNK)], wbuf)
            for g in range(CHUNK // L):
                sl = pl.ds(g * L, L)
                dr = plsc.load_gather(dinv, [ridx[sl]])
                dc = plsc.load_gather(dinv, [cidx[sl]])
                nbuf[sl] = wbuf[sl] * dr * dc
            pltpu.sync_copy(nbuf, norm_hbm.at[pl.ds(base, CHUNK)])
        return 0
    lax.fori_loop(0, LOOPS, body, 0)


@functools.partial(
    pl.kernel,
    out_type=jax.ShapeDtypeStruct((NC, N_NODES, D), jnp.float32),
    mesh=_sc_mesh,
    compiler_params=pltpu.CompilerParams(needs_layout_passes=False),
    scratch_types=[
        pltpu.VMEM((CHUNK,), jnp.int32),
        pltpu.VMEM((CHUNK,), jnp.int32),
        pltpu.VMEM((CHUNK,), jnp.float32),
        pltpu.VMEM((CHUNK, D), jnp.float32),
        pltpu.VMEM_SHARED((N_NODES, D), jnp.float32),
        pltpu.SemaphoreType.DMA,
    ],
)
def _msg_kernel(h_hbm, row_hbm, col_hbm, norm_hbm, out_hbm,
                ridx, cidx, nrm, rows, agg, sem):
    c = lax.axis_index("c")
    s = lax.axis_index("s")
    w = _worker_id()

    def zb(i, _):
        def zf(f, _):
            rows[i, pl.ds(f * L, L)] = jnp.zeros((L,), jnp.float32)
            return 0
        lax.fori_loop(0, D // L, zf, 0)
        return 0
    lax.fori_loop(0, CHUNK, zb, 0)
    base_row = s * 624
    for k in range(5):
        n = 128 if k < 4 else 624 - 4 * 128
        pltpu.sync_copy(rows.at[pl.ds(0, n)],
                        agg.at[pl.ds(base_row + k * 128, n)])
    @pl.when(s == NS - 1)
    def _():
        pltpu.sync_copy(rows.at[pl.ds(0, 16)], agg.at[pl.ds(9984, 16)])
    plsc.subcore_barrier()

    def body(k, _):
        ch = w + k * NW
        @pl.when(ch < N_CHUNKS)
        def _():
            base = ch * CHUNK
            pltpu.sync_copy(row_hbm.at[pl.ds(base, CHUNK)], ridx)
            pltpu.sync_copy(col_hbm.at[pl.ds(base, CHUNK)], cidx)
            pltpu.sync_copy(norm_hbm.at[pl.ds(base, CHUNK)], nrm)
            pltpu.async_copy(h_hbm.at[ridx], rows, sem).wait()

            def g_body(g, _):
                nv = nrm[pl.ds(g * L, L)]
                for e in range(L):
                    sc_ = nv[e]
                    for f in range(D // L):
                        sl = pl.ds(f * L, L)
                        rows[g * L + e, sl] = rows[g * L + e, sl] * sc_
                return 0
            lax.fori_loop(0, CHUNK // L, g_body, 0)
            pltpu.sync_copy(rows, agg.at[cidx], add=True)
        return 0
    lax.fori_loop(0, LOOPS, body, 0)
    plsc.subcore_barrier()

    for k in range(5):
        n = 128 if k < 4 else 624 - 4 * 128
        pltpu.sync_copy(agg.at[pl.ds(base_row + k * 128, n)],
                        rows.at[pl.ds(0, n)])
        pltpu.sync_copy(rows.at[pl.ds(0, n)],
                        out_hbm.at[c, pl.ds(base_row + k * 128, n)])
    @pl.when(s == NS - 1)
    def _():
        pltpu.sync_copy(agg.at[pl.ds(9984, 16)], rows.at[pl.ds(0, 16)])
        pltpu.sync_copy(rows.at[pl.ds(0, 16)], out_hbm.at[c, pl.ds(9984, 16)])


_BLK = 1000
_GRID = N_NODES // _BLK


def _dot(a, b):
    return jnp.dot(a, b, preferred_element_type=jnp.float32,
                   precision=lax.Precision.HIGHEST)


def _tc_in_body(x_ref, w_ref, b_ref, o_ref):
    o_ref[...] = jnp.maximum(_dot(x_ref[...], w_ref[...]) + b_ref[...], 0.0)


_tc_in = pl.pallas_call(
    _tc_in_body,
    grid=(_GRID,),
    in_specs=[
        pl.BlockSpec((_BLK, D), lambda i: (i, 0)),
        pl.BlockSpec((D, D), lambda i: (0, 0)),
        pl.BlockSpec((1, D), lambda i: (0, 0)),
    ],
    out_specs=pl.BlockSpec((_BLK, D), lambda i: (i, 0)),
    out_shape=jax.ShapeDtypeStruct((N_NODES, D), jnp.float32),
)


def _tc_combine_body(p_ref, h_ref, h0_ref, w_ref, o_ref):
    a = (A_AGG * (p_ref[0] + p_ref[1]) + RW * h_ref[...]
         + BETA * h0_ref[...])
    o_ref[...] = jnp.maximum(_dot(a, w_ref[...]), 0.0)


_tc_combine = pl.pallas_call(
    _tc_combine_body,
    grid=(_GRID,),
    in_specs=[
        pl.BlockSpec((NC, _BLK, D), lambda i: (0, i, 0)),
        pl.BlockSpec((_BLK, D), lambda i: (i, 0)),
        pl.BlockSpec((_BLK, D), lambda i: (i, 0)),
        pl.BlockSpec((D, D), lambda i: (0, 0)),
    ],
    out_specs=pl.BlockSpec((_BLK, D), lambda i: (i, 0)),
    out_shape=jax.ShapeDtypeStruct((N_NODES, D), jnp.float32),
)


def _tc_out_body(p_ref, h_ref, h0_ref, w4_ref, wo_ref, bo_ref, o_ref):
    a = (A_AGG * (p_ref[0] + p_ref[1]) + RW * h_ref[...]
         + BETA * h0_ref[...])
    h4 = jnp.maximum(_dot(a, w4_ref[...]), 0.0)
    logits = _dot(h4, wo_ref[...]) + bo_ref[...]
    m = jnp.max(logits, axis=1, keepdims=True)
    ex = jnp.exp(logits - m)
    lse = jnp.log(jnp.sum(ex, axis=1, keepdims=True)) + m
    o_ref[...] = logits - lse


_tc_out = pl.pallas_call(
    _tc_out_body,
    grid=(_GRID,),
    in_specs=[
        pl.BlockSpec((NC, _BLK, D), lambda i: (0, i, 0)),
        pl.BlockSpec((_BLK, D), lambda i: (i, 0)),
        pl.BlockSpec((_BLK, D), lambda i: (i, 0)),
        pl.BlockSpec((D, D), lambda i: (0, 0)),
        pl.BlockSpec((D, D), lambda i: (0, 0)),
        pl.BlockSpec((1, D), lambda i: (0, 0)),
    ],
    out_specs=pl.BlockSpec((_BLK, D), lambda i: (i, 0)),
    out_shape=jax.ShapeDtypeStruct((N_NODES, D), jnp.float32),
)


def _tc_lc_body(wg_ref, o_ref):
    r = lax.broadcasted_iota(jnp.int32, (D, D), 0)
    col = lax.broadcasted_iota(jnp.int32, (D, D), 1)
    eye = jnp.where(r == col, 1.0, 0.0).astype(jnp.float32)
    total = jnp.float32(0.0)
    for i in range(NUM_LAYERS):
        diff = wg_ref[i] - eye
        total = total + jnp.sqrt(jnp.sum(diff * diff))
    o_ref[...] = jnp.full((1, 1), total, jnp.float32)


_tc_lc = pl.pallas_call(
    _tc_lc_body,
    out_shape=jax.ShapeDtypeStruct((1, 1), jnp.float32),
)


def kernel(x, edge_index, edge_weight, W_in, b_in, W_gcn, W_out, b_out):
    row = edge_index[0]
    col = edge_index[1]

    h0 = _tc_in(x, W_in, b_in.reshape(1, D))
    degp = _deg_kernel(row, edge_weight)
    norm = _norm_kernel(degp, row, col, edge_weight)

    h = h0
    for i in range(NUM_LAYERS - 1):
        p = _msg_kernel(h, row, col, norm)
        h = _tc_combine(p, h, h0, W_gcn[i])

    p = _msg_kernel(h, row, col, norm)
    wo_pad = jnp.zeros((D, D), jnp.float32).at[:, :NUM_CLASSES].set(W_out)
    bo_pad = jnp.full((D,), -1e30, jnp.float32).at[:NUM_CLASSES].set(b_out)
    yfull = _tc_out(p, h, h0, W_gcn[NUM_LAYERS - 1], wo_pad,
                    bo_pad.reshape(1, D))
    y = yfull[:, :NUM_CLASSES]

    lc = _tc_lc(W_gcn)[0, 0] * GAMMA
    return (y, lc)


# speedup vs baseline: 20.1735x; 2.3380x over previous
import functools
import math

import jax
import jax.numpy as jnp
from jax import lax
from jax.experimental import pallas as pl
from jax.experimental.pallas import tpu as pltpu
from jax.experimental.pallas import tpu_sc as plsc

N_NODES = 10000
N_PAD = 10240
N_EDGES = 320000
D = 128
NUM_CLASSES = 40
NUM_LAYERS = 4
C_MIN = 0.2
C_MAX = 1.0
BETA = 0.1
GAMMA = 1.0
RW = C_MIN - BETA
A_AGG = 1.0 - RW - BETA

NC = 2
NS = 16
NW = NC * NS
L = 16
CHUNK = 128
N_CHUNKS = N_EDGES // CHUNK
CPW = N_CHUNKS // NW
LEFT0 = CPW * NW
N_LEFT = N_CHUNKS - LEFT0
P = CPW // 2
SLAB = P + 1

_sc_mesh = plsc.VectorSubcoreMesh(
    core_axis_name="c", subcore_axis_name="s", num_cores=NC, num_subcores=NS)


def _worker_id():
    return lax.axis_index("s") * NC + lax.axis_index("c")


@functools.partial(
    pl.kernel,
    out_type=jax.ShapeDtypeStruct((NC * N_PAD,), jnp.float32),
    mesh=_sc_mesh,
    compiler_params=pltpu.CompilerParams(needs_layout_passes=False),
    scratch_types=[
        pltpu.VMEM((CPW + 1, 1, CHUNK), jnp.int32),
        pltpu.VMEM((CPW + 1, 1, CHUNK), jnp.float32),
        pltpu.VMEM((1024,), jnp.float32),
        pltpu.VMEM_SHARED((N_NODES,), jnp.float32),
        pltpu.SemaphoreType.DMA,
    ],
)
def _deg_kernel(row3_hbm, w3_hbm, deg_hbm, ridx3, wsl, zbuf, deg_sp, dsem):
    c = lax.axis_index("c")
    s = lax.axis_index("s")
    w = _worker_id()
    base = w * CPW

    pltpu.sync_copy(row3_hbm.at[pl.ds(base, CPW)], ridx3.at[pl.ds(0, CPW)])
    pltpu.sync_copy(w3_hbm.at[pl.ds(base, CPW)], wsl.at[pl.ds(0, CPW)])
    @pl.when(w < N_LEFT)
    def _():
        pltpu.sync_copy(row3_hbm.at[pl.ds(LEFT0 + w, 1)],
                        ridx3.at[pl.ds(CPW, 1)])
        pltpu.sync_copy(w3_hbm.at[pl.ds(LEFT0 + w, 1)],
                        wsl.at[pl.ds(CPW, 1)])

    def zb(i, _):
        zbuf[pl.ds(i * L, L)] = jnp.zeros((L,), jnp.float32)
        return 0
    lax.fori_loop(0, 1024 // L, zb, 0)
    @pl.when(s < 10)
    def _():
        pltpu.sync_copy(zbuf.at[pl.ds(0, 1000)], deg_sp.at[pl.ds(s * 1000, 1000)])
    plsc.subcore_barrier()

    def body(k, _):
        pltpu.async_copy(wsl.at[k, 0], deg_sp.at[ridx3.at[k, 0]], dsem,
                         add=True)
        return 0
    lax.fori_loop(0, CPW, body, 0)
    def drain(k, _):
        pltpu.make_async_copy(wsl.at[0, 0], deg_sp.at[ridx3.at[0, 0]],
                              dsem).wait()
        return 0
    lax.fori_loop(0, CPW, drain, 0)
    @pl.when(w < N_LEFT)
    def _():
        pltpu.sync_copy(wsl.at[CPW, 0], deg_sp.at[ridx3.at[CPW, 0]], add=True)
    plsc.subcore_barrier()

    @pl.when(s < 10)
    def _():
        pltpu.sync_copy(deg_sp.at[pl.ds(s * 1000, 1000)], zbuf.at[pl.ds(0, 1000)])
        pltpu.sync_copy(zbuf.at[pl.ds(0, 1000)],
                        deg_hbm.at[pl.ds(c * N_PAD + s * 1000, 1000)])
    @pl.when(s == 10)
    def _():
        def zz(i, _):
            zbuf[pl.ds(i * L, L)] = jnp.zeros((L,), jnp.float32)
            return 0
        lax.fori_loop(0, 240 // L, zz, 0)
        pltpu.sync_copy(zbuf.at[pl.ds(0, 240)],
                        deg_hbm.at[pl.ds(c * N_PAD + N_NODES, 240)])


@functools.partial(
    pl.kernel,
    out_type=jax.ShapeDtypeStruct((NC, N_NODES, D), jnp.float32),
    mesh=_sc_mesh,
    compiler_params=pltpu.CompilerParams(needs_layout_passes=False),
    scratch_types=[
        pltpu.VMEM((SLAB, 1, CHUNK), jnp.int32),
        pltpu.VMEM((SLAB, 1, CHUNK), jnp.int32),
        pltpu.VMEM((SLAB, 1, CHUNK), jnp.float32),
        pltpu.VMEM((2, CHUNK, D), jnp.float32),
        pltpu.VMEM_SHARED((N_NODES, D), jnp.float32),
        [pltpu.SemaphoreType.DMA] * 2,
        [pltpu.SemaphoreType.DMA] * 2,
    ],
)
def _msg_kernel(hd_hbm, row3_hbm, col3_hbm, w3_hbm, out_hbm,
                ridx3, cidx3, wsl, rows, agg, gsems, ssems):
    c = lax.axis_index("c")
    s = lax.axis_index("s")
    w = _worker_id()
    base = w * CPW

    def load_slabs(pass_idx, with_leftover):
        pbase = base + pass_idx * P
        pltpu.sync_copy(row3_hbm.at[pl.ds(pbase, P)], ridx3.at[pl.ds(0, P)])
        pltpu.sync_copy(col3_hbm.at[pl.ds(pbase, P)], cidx3.at[pl.ds(0, P)])
        pltpu.sync_copy(w3_hbm.at[pl.ds(pbase, P)], wsl.at[pl.ds(0, P)])
        if with_leftover:
            @pl.when(w < N_LEFT)
            def _():
                pltpu.sync_copy(row3_hbm.at[pl.ds(LEFT0 + w, 1)],
                                ridx3.at[pl.ds(P, 1)])
                pltpu.sync_copy(col3_hbm.at[pl.ds(LEFT0 + w, 1)],
                                cidx3.at[pl.ds(P, 1)])
                pltpu.sync_copy(w3_hbm.at[pl.ds(LEFT0 + w, 1)],
                                wsl.at[pl.ds(P, 1)])

    load_slabs(0, False)

    def zb(i, _):
        def zf(f, _):
            rows[0, i, pl.ds(f * L, L)] = jnp.zeros((L,), jnp.float32)
            return 0
        lax.fori_loop(0, D // L, zf, 0)
        return 0
    lax.fori_loop(0, CHUNK, zb, 0)
    base_row = s * 624
    for k in range(5):
        n = 128 if k < 4 else 624 - 4 * 128
        pltpu.sync_copy(rows.at[0, pl.ds(0, n)],
                        agg.at[pl.ds(base_row + k * 128, n)])
    @pl.when(s == NS - 1)
    def _():
        pltpu.sync_copy(rows.at[0, pl.ds(0, 16)], agg.at[pl.ds(9984, 16)])
    plsc.subcore_barrier()

    def issue_gather(kk, b):
        pltpu.async_copy(hd_hbm.at[ridx3.at[kk, 0]], rows.at[b], gsems[b])

    def wait_gather(kk, b):
        pltpu.make_async_copy(hd_hbm.at[ridx3.at[kk, 0]], rows.at[b],
                              gsems[b]).wait()

    def issue_scatter(kk, b):
        pltpu.async_copy(rows.at[b], agg.at[cidx3.at[kk, 0]], ssems[b],
                         add=True)

    def wait_scatter(b):
        pltpu.make_async_copy(rows.at[b], agg.at[cidx3.at[0, 0]],
                              ssems[b]).wait()

    def scale(kk, b):
        def g_body(g, _):
            sv = wsl[kk, 0, pl.ds(g * L, L)]
            for e in range(L):
                f = sv[e]
                for q in range(D // L):
                    slq = pl.ds(q * L, L)
                    rows[b, g * L + e, slq] = rows[b, g * L + e, slq] * f
            return 0
        lax.fori_loop(0, CHUNK // L, g_body, 0)

    def run_pass():
        issue_gather(0, 0)
        issue_gather(1, 1)
        wait_gather(0, 0)
        scale(0, 0)
        issue_scatter(0, 0)
        wait_scatter(0)
        issue_gather(2, 0)
        wait_gather(1, 1)
        scale(1, 1)
        issue_scatter(1, 1)

        def group(t, _):
            for b in range(2):
                kk = 2 * t + b
                wait_scatter(1 - b)
                issue_gather(kk + 1, 1 - b)
                wait_gather(kk, b)
                scale(kk, b)
                issue_scatter(kk, b)
            return 0
        lax.fori_loop(1, P // 2, group, 0)

        wait_scatter(1)
        wait_gather(P - 1, 0)
        scale(P - 1, 0)
        issue_scatter(P - 1, 0)
        wait_scatter(0)

    run_pass()
    load_slabs(1, True)
    run_pass()

    @pl.when(w < N_LEFT)
    def _():
        issue_gather(P, 1)
        wait_gather(P, 1)
        scale(P, 1)
        issue_scatter(P, 1)
        wait_scatter(1)

    plsc.subcore_barrier()

    for k in range(5):
        n = 128 if k < 4 else 624 - 4 * 128
        pltpu.sync_copy(agg.at[pl.ds(base_row + k * 128, n)],
                        rows.at[0, pl.ds(0, n)])
        pltpu.sync_copy(rows.at[0, pl.ds(0, n)],
                        out_hbm.at[c, pl.ds(base_row + k * 128, n)])
    @pl.when(s == NS - 1)
    def _():
        pltpu.sync_copy(agg.at[pl.ds(9984, 16)], rows.at[0, pl.ds(0, 16)])
        pltpu.sync_copy(rows.at[0, pl.ds(0, 16)], out_hbm.at[c, pl.ds(9984, 16)])


_BLK = 1000
_GRID = N_NODES // _BLK


def _dot(a, b):
    return jnp.dot(a, b, preferred_element_type=jnp.float32,
                   precision=lax.Precision.HIGHEST)


def _tc_in_body(x_ref, w_ref, b_ref, o_ref):
    o_ref[...] = jnp.maximum(_dot(x_ref[...], w_ref[...]) + b_ref[...], 0.0)


_tc_in = pl.pallas_call(
    _tc_in_body,
    grid=(_GRID,),
    in_specs=[
        pl.BlockSpec((_BLK, D), lambda i: (i, 0)),
        pl.BlockSpec((D, D), lambda i: (0, 0)),
        pl.BlockSpec((1, D), lambda i: (0, 0)),
    ],
    out_specs=pl.BlockSpec((_BLK, D), lambda i: (i, 0)),
    out_shape=jax.ShapeDtypeStruct((N_NODES, D), jnp.float32),
)


def _tc_dinv_body(dp_ref, o_ref):
    d = dp_ref[0] + dp_ref[1]
    o_ref[...] = jnp.where(d > 0.0, lax.rsqrt(d), 0.0)


_tc_dinv = pl.pallas_call(
    _tc_dinv_body,
    in_specs=[pl.BlockSpec((NC, N_PAD // D, D), lambda: (0, 0, 0))],
    out_specs=pl.BlockSpec((N_PAD // D, D), lambda: (0, 0)),
    out_shape=jax.ShapeDtypeStruct((N_PAD // D, D), jnp.float32),
)


def _tc_hd0_body(dv_ref, h_ref, o_ref):
    o_ref[...] = dv_ref[...] * h_ref[...]


_tc_hd0 = pl.pallas_call(
    _tc_hd0_body,
    grid=(_GRID,),
    in_specs=[
        pl.BlockSpec((_BLK, 1), lambda i: (i, 0)),
        pl.BlockSpec((_BLK, D), lambda i: (i, 0)),
    ],
    out_specs=pl.BlockSpec((_BLK, D), lambda i: (i, 0)),
    out_shape=jax.ShapeDtypeStruct((N_NODES, D), jnp.float32),
)


def _tc_combine_body(p_ref, dv_ref, h_ref, h0_ref, w_ref, o_ref, od_ref):
    a = (A_AGG * (p_ref[0] + p_ref[1]) * dv_ref[...] + RW * h_ref[...]
         + BETA * h0_ref[...])
    hn = jnp.maximum(_dot(a, w_ref[...]), 0.0)
    o_ref[...] = hn
    od_ref[...] = hn * dv_ref[...]


_tc_combine = pl.pallas_call(
    _tc_combine_body,
    grid=(_GRID,),
    in_specs=[
        pl.BlockSpec((NC, _BLK, D), lambda i: (0, i, 0)),
        pl.BlockSpec((_BLK, 1), lambda i: (i, 0)),
        pl.BlockSpec((_BLK, D), lambda i: (i, 0)),
        pl.BlockSpec((_BLK, D), lambda i: (i, 0)),
        pl.BlockSpec((D, D), lambda i: (0, 0)),
    ],
    out_specs=[
        pl.BlockSpec((_BLK, D), lambda i: (i, 0)),
        pl.BlockSpec((_BLK, D), lambda i: (i, 0)),
    ],
    out_shape=[
        jax.ShapeDtypeStruct((N_NODES, D), jnp.float32),
        jax.ShapeDtypeStruct((N_NODES, D), jnp.float32),
    ],
)


def _tc_out_body(p_ref, dv_ref, h_ref, h0_ref, w4_ref, wo_ref, bo_ref, o_ref):
    a = (A_AGG * (p_ref[0] + p_ref[1]) * dv_ref[...] + RW * h_ref[...]
         + BETA * h0_ref[...])
    h4 = jnp.maximum(_dot(a, w4_ref[...]), 0.0)
    logits = _dot(h4, wo_ref[...]) + bo_ref[...]
    m = jnp.max(logits, axis=1, keepdims=True)
    ex = jnp.exp(logits - m)
    lse = jnp.log(jnp.sum(ex, axis=1, keepdims=True)) + m
    o_ref[...] = logits - lse


_tc_out = pl.pallas_call(
    _tc_out_body,
    grid=(_GRID,),
    in_specs=[
        pl.BlockSpec((NC, _BLK, D), lambda i: (0, i, 0)),
        pl.BlockSpec((_BLK, 1), lambda i: (i, 0)),
        pl.BlockSpec((_BLK, D), lambda i: (i, 0)),
        pl.BlockSpec((_BLK, D), lambda i: (i, 0)),
        pl.BlockSpec((D, D), lambda i: (0, 0)),
        pl.BlockSpec((D, D), lambda i: (0, 0)),
        pl.BlockSpec((1, D), lambda i: (0, 0)),
    ],
    out_specs=pl.BlockSpec((_BLK, D), lambda i: (i, 0)),
    out_shape=jax.ShapeDtypeStruct((N_NODES, D), jnp.float32),
)


def _tc_lc_body(wg_ref, o_ref):
    r = lax.broadcasted_iota(jnp.int32, (D, D), 0)
    col = lax.broadcasted_iota(jnp.int32, (D, D), 1)
    eye = jnp.where(r == col, 1.0, 0.0).astype(jnp.float32)
    total = jnp.float32(0.0)
    for i in range(NUM_LAYERS):
        diff = wg_ref[i] - eye
        total = total + jnp.sqrt(jnp.sum(diff * diff))
    o_ref[...] = jnp.full((1, 1), total, jnp.float32)


_tc_lc = pl.pallas_call(
    _tc_lc_body,
    out_shape=jax.ShapeDtypeStruct((1, 1), jnp.float32),
)


def kernel(x, edge_index, edge_weight, W_in, b_in, W_gcn, W_out, b_out):
    row3 = edge_index[0].reshape(N_CHUNKS, 1, CHUNK)
    col3 = edge_index[1].reshape(N_CHUNKS, 1, CHUNK)
    w3 = edge_weight.reshape(N_CHUNKS, 1, CHUNK)

    h0 = _tc_in(x, W_in, b_in.reshape(1, D))
    degp = _deg_kernel(row3, w3)
    dinv2d = _tc_dinv(degp.reshape(NC, N_PAD // D, D))
    dinv_col = dinv2d.reshape(N_PAD)[:N_NODES].reshape(N_NODES, 1)

    h = h0
    hd = _tc_hd0(dinv_col, h0)
    for i in range(NUM_LAYERS - 1):
        p = _msg_kernel(hd, row3, col3, w3)
        h, hd = _tc_combine(p, dinv_col, h, h0, W_gcn[i])

    p = _msg_kernel(hd, row3, col3, w3)
    wo_pad = jnp.zeros((D, D), jnp.float32).at[:, :NUM_CLASSES].set(W_out)
    bo_pad = jnp.full((D,), -1e30, jnp.float32).at[:NUM_CLASSES].set(b_out)
    yfull = _tc_out(p, dinv_col, h, h0, W_gcn[NUM_LAYERS - 1], wo_pad,
                    bo_pad.reshape(1, D))
    y = yfull[:, :NUM_CLASSES]

    lc = _tc_lc(W_gcn)[0, 0] * GAMMA
    return (y, lc)
